# Initial kernel scaffold; baseline (speedup 1.0000x reference)
import jax
import jax.numpy as jnp
from jax.experimental import pallas as pl


def kernel(codes, tables, mask_emb, W, b):
    raise NotImplementedError("write your pallas kernel here")


# speedup vs baseline: 2.0584x; 2.0584x over previous
import functools

import jax
import jax.numpy as jnp
from jax import lax
from jax.experimental import pallas as pl
from jax.experimental.pallas import tpu as pltpu
from jax.experimental.pallas import tpu_sc as plsc

N_CB = 14
VOCAB = 1024
D = 1024
MASK = 1024
ROWS = VOCAB + 1
BATCH = 4
SEQ = 2048
NPOS = BATCH * SEQ
BV = 128
CH = 8


def _proj_body(t_ref, me_ref, w_ref, b_ref, p_ref):
    i = pl.program_id(0)
    vb = pl.program_id(1)
    t = t_ref[0]
    rows = vb * BV + lax.broadcasted_iota(jnp.int32, (BV, 1), 0)
    t = jnp.where(rows == MASK, me_ref[0], t)
    acc = lax.dot_general(t, w_ref[...], (((1,), (1,)), ((), ())),
                          preferred_element_type=jnp.float32)
    p_ref[0] = acc + jnp.where(i == 0, b_ref[...], 0.0)


def _project(tables, mask_emb, W, b2d):
    nvb = (ROWS + BV - 1) // BV
    return pl.pallas_call(
        _proj_body,
        grid=(N_CB, nvb),
        in_specs=[
            pl.BlockSpec((1, BV, D), lambda i, v: (i, v, 0)),
            pl.BlockSpec((1, 1, D), lambda i, v: (i, 0, 0)),
            pl.BlockSpec((D, D), lambda i, v: (0, i)),
            pl.BlockSpec((1, D), lambda i, v: (0, 0)),
        ],
        out_specs=pl.BlockSpec((1, BV, D), lambda i, v: (i, v, 0)),
        out_shape=jax.ShapeDtypeStruct((N_CB, ROWS, D), jnp.float32),
    )(tables, mask_emb.reshape(N_CB, 1, D), W, b2d)


def _sc_body(codes_hbm, p_hbm, out_hbm, idx_v, gbuf, accv, gsem):
    info = plsc.get_sparse_core_info()
    nc = info.num_cores
    wid = lax.axis_index("s") * nc + lax.axis_index("c")
    per_w = NPOS // (nc * info.num_subcores)
    base = wid * per_w
    b_idx = base // SEQ
    s0 = base % SEQ

    for i in range(N_CB):
        pltpu.sync_copy(codes_hbm.at[b_idx * N_CB + i, pl.ds(s0, per_w)],
                        idx_v.at[i])
    for i in range(N_CB):
        def _off_body(j, _, i=i):
            idx_v[i, pl.ds(j * 16, 16)] = idx_v[i, pl.ds(j * 16, 16)] + i * ROWS
            return 0
        lax.fori_loop(0, per_w // 16, _off_body, 0)

    nchunks = per_w // CH

    def _chunk(j, _):
        descs = []
        for i in range(N_CB):
            descs.append(pltpu.async_copy(
                p_hbm.at[idx_v.at[i, pl.ds(j * CH, CH)]], gbuf.at[i], gsem))
        for d in descs:
            d.wait()
        for p in range(CH):
            def _col(v, _, p=p):
                sl = pl.ds(v * 16, 16)
                acc = gbuf[0, p, sl]
                for i in range(1, N_CB):
                    acc = acc + gbuf[i, p, sl]
                accv[p, sl] = acc
                return 0
            lax.fori_loop(0, D // 16, _col, 0)
        pltpu.sync_copy(accv, out_hbm.at[pl.ds(base + j * CH, CH)])
        return 0

    lax.fori_loop(0, nchunks, _chunk, 0)


def _gather_sum(codes2, p_flat):
    mesh = plsc.VectorSubcoreMesh(core_axis_name="c", subcore_axis_name="s")
    f = functools.partial(
        pl.kernel,
        mesh=mesh,
        out_type=jax.ShapeDtypeStruct((NPOS, D), jnp.float32),
        scratch_types=[
            pltpu.VMEM((N_CB, 256), jnp.int32),
            pltpu.VMEM((N_CB, CH, D), jnp.float32),
            pltpu.VMEM((CH, D), jnp.float32),
            pltpu.SemaphoreType.DMA,
        ],
    )(_sc_body)
    return f(codes2, p_flat)


def kernel(codes, tables, mask_emb, W, b):
    P = _project(tables, mask_emb, W, b.reshape(1, D))
    p_flat = P.reshape(N_CB * ROWS, D)
    codes2 = codes.reshape(BATCH * N_CB, SEQ).astype(jnp.int32)
    out = _gather_sum(codes2, p_flat)
    return out.reshape(BATCH, SEQ, D)


# speedup vs baseline: 2.1410x; 1.0401x over previous
import functools

import jax
import jax.numpy as jnp
from jax import lax
from jax.experimental import pallas as pl
from jax.experimental.pallas import tpu as pltpu
from jax.experimental.pallas import tpu_sc as plsc

N_CB = 14
VOCAB = 1024
D = 1024
MASK = 1024
ROWS = VOCAB + 1
BATCH = 4
SEQ = 2048
NPOS = BATCH * SEQ
BV = 128
CH = 8


def _proj_body(t_ref, me_ref, w_ref, b_ref, p_ref):
    i = pl.program_id(0)
    vb = pl.program_id(1)
    t = t_ref[0]
    rows = vb * BV + lax.broadcasted_iota(jnp.int32, (BV, 1), 0)
    t = jnp.where(rows == MASK, me_ref[0], t)
    acc = lax.dot_general(t, w_ref[...], (((1,), (1,)), ((), ())),
                          preferred_element_type=jnp.float32)
    p_ref[0] = acc + jnp.where(i == 0, b_ref[...], 0.0)


def _project(tables, mask_emb, W, b2d):
    nvb = (ROWS + BV - 1) // BV
    return pl.pallas_call(
        _proj_body,
        grid=(N_CB, nvb),
        in_specs=[
            pl.BlockSpec((1, BV, D), lambda i, v: (i, v, 0)),
            pl.BlockSpec((1, 1, D), lambda i, v: (i, 0, 0)),
            pl.BlockSpec((D, D), lambda i, v: (0, i)),
            pl.BlockSpec((1, D), lambda i, v: (0, 0)),
        ],
        out_specs=pl.BlockSpec((1, BV, D), lambda i, v: (i, v, 0)),
        out_shape=jax.ShapeDtypeStruct((N_CB, ROWS, D), jnp.float32),
    )(tables, mask_emb.reshape(N_CB, 1, D), W, b2d)


def _sc_body(codes_hbm, p_hbm, out_hbm, idx_v, gbuf, accv,
             gsem0, gsem1, osem):
    info = plsc.get_sparse_core_info()
    nc = info.num_cores
    wid = lax.axis_index("s") * nc + lax.axis_index("c")
    per_w = NPOS // (nc * info.num_subcores)
    base = wid * per_w
    b_idx = base // SEQ
    s0 = base % SEQ

    for i in range(N_CB):
        pltpu.sync_copy(codes_hbm.at[b_idx * N_CB + i, pl.ds(s0, per_w)],
                        idx_v.at[i])
    for i in range(N_CB):
        def _off_body(j, _, i=i):
            idx_v[i, pl.ds(j * 16, 16)] = idx_v[i, pl.ds(j * 16, 16)] + i * ROWS
            return 0
        lax.fori_loop(0, per_w // 16, _off_body, 0)

    nchunks = per_w // CH
    half = N_CB // 2
    gsems = (gsem0, gsem1)

    def _fire(j, g):
        for k in range(half):
            i = g * half + k
            pltpu.async_copy(p_hbm.at[idx_v.at[i, pl.ds(j * CH, CH)]],
                             gbuf.at[g, k], gsems[g])

    def _drain_gather(g):
        for k in range(half):
            pltpu.make_async_copy(p_hbm.at[pl.ds(0, CH)], gbuf.at[g, k],
                                  gsems[g]).wait()

    def _drain_out():
        pltpu.make_async_copy(accv, out_hbm.at[pl.ds(base, CH)], osem).wait()

    _fire(0, 0)
    _fire(0, 1)

    def _chunk(j, _):
        for g in range(2):
            _drain_gather(g)
            if g == 0:
                @pl.when(j > 0)
                def _():
                    _drain_out()
            for p in range(CH):
                def _col(v, _, p=p, g=g):
                    sl = pl.ds(v * 16, 16)
                    acc = gbuf[g, 0, p, sl] if g == 0 else accv[p, sl]
                    for k in range(0 if g else 1, half):
                        acc = acc + gbuf[g, k, p, sl]
                    accv[p, sl] = acc
                    return 0
                lax.fori_loop(0, D // 16, _col, 0)

            @pl.when(j + 1 < nchunks)
            def _(g=g):
                _fire(j + 1, g)
        pltpu.async_copy(accv, out_hbm.at[pl.ds(base + j * CH, CH)], osem)
        return 0

    lax.fori_loop(0, nchunks, _chunk, 0)
    _drain_out()


def _gather_sum(codes2, p_flat):
    mesh = plsc.VectorSubcoreMesh(core_axis_name="c", subcore_axis_name="s")
    f = functools.partial(
        pl.kernel,
        mesh=mesh,
        out_type=jax.ShapeDtypeStruct((NPOS, D), jnp.float32),
        scratch_types=[
            pltpu.VMEM((N_CB, 256), jnp.int32),
            pltpu.VMEM((2, N_CB // 2, CH, D), jnp.float32),
            pltpu.VMEM((CH, D), jnp.float32),
            pltpu.SemaphoreType.DMA,
            pltpu.SemaphoreType.DMA,
            pltpu.SemaphoreType.DMA,
        ],
    )(_sc_body)
    return f(codes2, p_flat)


def kernel(codes, tables, mask_emb, W, b):
    P = _project(tables, mask_emb, W, b.reshape(1, D))
    p_flat = P.reshape(N_CB * ROWS, D)
    codes2 = codes.reshape(BATCH * N_CB, SEQ).astype(jnp.int32)
    out = _gather_sum(codes2, p_flat)
    return out.reshape(BATCH, SEQ, D)
